# baseline (speedup 1.0000x reference)
import functools

import jax
import jax.numpy as jnp
from jax import lax
from jax.experimental import pallas as pl
from jax.experimental.pallas import tpu as pltpu
from jax.experimental.pallas import tpu_sc as plsc

N_ANCHORS = 76725
N_PAD = 76800
ROWS = 600
TOP_K = 1000
MAX_DET = 100
IOU_THR = 0.5
SCORE_THR = 0.5
IMG_W = 640.0
IMG_H = 640.0
NEG_INF = float("-inf")
INT_MIN = -2147483648

N_TILES = 16
CHUNK = N_PAD // N_TILES
N_FIELDS = 6
COMPACT = 1024


def _decode_kernel(pred_ref, anc_ref, vals_out, key_out, thr_out):
    f32 = jnp.float32
    i32 = jnp.int32

    ridx = lax.broadcasted_iota(i32, (ROWS, 128), 0)
    cidx = lax.broadcasted_iota(i32, (ROWS, 128), 1)
    gidx = ridx * 128 + cidx

    c0 = pred_ref[4]
    c1 = pred_ref[5]
    score = jnp.maximum(c0, c1) + 0.0
    idf = (c1 > c0).astype(f32)

    ibits = lax.bitcast_convert_type(score, i32)
    key = jnp.where(ibits >= 0, ibits, ibits ^ jnp.int32(0x7FFFFFFF))
    key = jnp.where(gidx < N_ANCHORS, key, jnp.int32(INT_MIN))

    b0 = pred_ref[0] * f32(0.1)
    b1 = pred_ref[1] * f32(0.1)
    b2 = pred_ref[2] * f32(0.2)
    b3 = pred_ref[3] * f32(0.2)
    ax = anc_ref[0]
    ay = anc_ref[1]
    aw = anc_ref[2]
    ah = anc_ref[3]
    x = b0 * aw + ax
    y = b1 * ah + ay
    w = jnp.exp(b2) * aw
    h = jnp.exp(b3) * ah
    vals_out[0] = jnp.clip(x - w / 2.0, 0.0, IMG_W)
    vals_out[1] = jnp.clip(y - h / 2.0, 0.0, IMG_H)
    vals_out[2] = jnp.clip(x + w / 2.0, 0.0, IMG_W)
    vals_out[3] = jnp.clip(y + h / 2.0, 0.0, IMG_H)
    vals_out[4] = score
    vals_out[5] = idf
    key_out[...] = key

    def count_ge(t):
        return jnp.sum((key >= t).astype(i32))

    cur0 = jnp.where(count_ge(jnp.int32(0)) >= TOP_K,
                     jnp.int32(0), jnp.int32(INT_MIN))

    def bit_step(t, cur):
        cand = cur + (jnp.int32(1) << (jnp.int32(30) - t))
        return jnp.where(count_ge(cand) >= TOP_K, cand, cur)

    t_key = lax.fori_loop(0, 31, bit_step, cur0)
    m_gt = jnp.sum((key > t_key).astype(i32))
    r_need = TOP_K - m_gt
    eq = key == t_key

    def idx_step(_, lohi):
        lo, hi = lohi
        mid = (lo + hi) // 2
        cnt = jnp.sum((eq & (gidx <= mid)).astype(i32))
        p = cnt >= r_need
        return jnp.where(p, lo, mid + 1), jnp.where(p, mid, hi)

    lo, _ = lax.fori_loop(0, 17, idx_step,
                          (jnp.int32(0), jnp.int32(N_PAD - 1)))

    trow = lax.broadcasted_iota(i32, (8, 128), 0)
    thr_out[...] = jnp.where(trow == 0, t_key, lo)


def _compact_kernel(key_hbm, vals_hbm, t_hbm, c_hbm,
                    o0, o1, o2, o3, o4, o5,
                    key_v, vals_v, cl_v, t_v, c_v, cnt_v, allcnt_v,
                    shared_cnt, sem):
    i32 = jnp.int32
    wid = lax.axis_index("s")
    base = wid * CHUNK
    outs = (o0, o1, o2, o3, o4, o5)

    cps = [pltpu.async_copy(key_hbm.at[pl.ds(base, CHUNK)], key_v, sem)]
    for jf in range(N_FIELDS):
        cps.append(
            pltpu.async_copy(vals_hbm.at[pl.ds(jf * N_PAD + base, CHUNK)],
                             vals_v.at[pl.ds(jf * CHUNK, CHUNK)], sem))
    cps.append(pltpu.async_copy(t_hbm, t_v, sem))
    cps.append(pltpu.async_copy(c_hbm, c_v, sem))
    for cp in cps:
        cp.wait()

    tvec = t_v[...]
    cvec = c_v[...]

    def body(i, n):
        k = key_v[pl.ds(i * 16, 16)]
        gi = base + i * 16 + lax.iota(i32, 16)
        sel = (k > tvec) | ((k == tvec) & (gi <= cvec))
        seli = sel.astype(i32)
        pos = n + plsc.cumsum(seli) - 1
        for jf in range(N_FIELDS):
            v = vals_v[pl.ds(jf * CHUNK + i * 16, 16)]
            plsc.store_scatter(cl_v, [pos + jf * (CHUNK + 16)], v, mask=sel)
        return n + jnp.sum(seli)

    n = lax.fori_loop(0, CHUNK // 16, body, jnp.int32(0))

    cnt_v[...] = jnp.full((16,), n, i32)
    pltpu.sync_copy(cnt_v, shared_cnt.at[pl.ds(wid * 16, 16)])
    plsc.subcore_barrier()
    pltpu.sync_copy(shared_cnt, allcnt_v)

    basev = jnp.zeros((16,), i32)
    for w in range(N_TILES):
        basev = basev + jnp.where(w < wid, allcnt_v[pl.ds(w * 16, 16)],
                                  jnp.zeros((16,), i32))

    nch = (n + 15) // 16

    def sbody(j, _):
        lpos = j * 16 + lax.iota(i32, 16)
        msk = lpos < n
        safe = jnp.where(msk, basev + lpos, TOP_K + lax.iota(i32, 16))
        for jf in range(N_FIELDS):
            pltpu.sync_copy(
                cl_v.at[pl.ds(jf * (CHUNK + 16) + j * 16, 16)],
                outs[jf].at[safe])
        return 0

    lax.fori_loop(0, nch, sbody, 0)


def _nms_kernel(cf_ref, s0, s1, s2, s3, s5, boxes_out, misc_out):
    i32 = jnp.int32
    ridx = lax.broadcasted_iota(i32, (8, 128), 0)
    cidx = lax.broadcasted_iota(i32, (8, 128), 1)
    pos = ridx * 128 + cidx

    x1 = cf_ref[0]
    y1 = cf_ref[1]
    x2 = cf_ref[2]
    y2 = cf_ref[3]
    s = cf_ref[4]
    ar = (x2 - x1) * (y2 - y1)
    w0 = jnp.where((pos < TOP_K) & (s >= SCORE_THR), s, NEG_INF)

    def nms_step(t, wv):
        m = jnp.max(wv)
        valid = m > NEG_INF
        eqm = wv == m
        i = jnp.min(jnp.where(eqm, pos, jnp.int32(2**30)))
        em = pos == i
        bx1 = s0[i]
        by1 = s1[i]
        bx2 = s2[i]
        by2 = s3[i]
        bid = s5[i]
        bar = (bx2 - bx1) * (by2 - by1)
        xx1 = jnp.maximum(bx1, x1)
        yy1 = jnp.maximum(by1, y1)
        xx2 = jnp.minimum(bx2, x2)
        yy2 = jnp.minimum(by2, y2)
        inter = jnp.maximum(xx2 - xx1, 0.0) * jnp.maximum(yy2 - yy1, 0.0)
        union = bar + ar - inter
        iou = inter / jnp.maximum(union, 1e-8)
        wnext = jnp.where((iou > IOU_THR) | em, NEG_INF, wv)

        vf = valid.astype(jnp.float32)
        row = jnp.concatenate(
            [jnp.where(valid, b, 0.0).reshape(1, 1)
             for b in (bx1, by1, bx2, by2)], axis=1)
        boxes_out[pl.ds(t, 1), :] = row
        mrow = jnp.concatenate([
            jnp.where(valid, bid, -1.0).reshape(1, 1),
            jnp.where(valid, m, 0.0).reshape(1, 1),
            vf.reshape(1, 1),
            jnp.zeros((1, 1), jnp.float32)], axis=1)
        misc_out[pl.ds(t, 1), :] = mrow
        return wnext

    lax.fori_loop(0, MAX_DET, nms_step, w0)


@jax.jit
def kernel(predictions, anchors):
    f32 = jnp.float32
    i32 = jnp.int32
    pred_t = jnp.pad(predictions[0].T, ((0, 0), (0, N_PAD - N_ANCHORS)))
    pred_t = pred_t.reshape(6, ROWS, 128)
    anc_t = jnp.pad(anchors.T, ((0, 0), (0, N_PAD - N_ANCHORS)))
    anc_t = anc_t.reshape(4, ROWS, 128)

    vals, key, thr = pl.pallas_call(
        _decode_kernel,
        out_shape=[
            jax.ShapeDtypeStruct((N_FIELDS, ROWS, 128), f32),
            jax.ShapeDtypeStruct((ROWS, 128), i32),
            jax.ShapeDtypeStruct((8, 128), i32),
        ],
    )(pred_t, anc_t)

    mesh = plsc.VectorSubcoreMesh(core_axis_name="c", subcore_axis_name="s",
                                  num_cores=1)
    compact = functools.partial(
        pl.kernel, mesh=mesh,
        compiler_params=pltpu.CompilerParams(needs_layout_passes=False),
        out_type=[jax.ShapeDtypeStruct((COMPACT,), f32)] * N_FIELDS,
        scratch_types=[
            pltpu.VMEM((CHUNK,), i32),
            pltpu.VMEM((N_FIELDS * CHUNK,), f32),
            pltpu.VMEM((N_FIELDS * (CHUNK + 16),), f32),
            pltpu.VMEM((16,), i32),
            pltpu.VMEM((16,), i32),
            pltpu.VMEM((16,), i32),
            pltpu.VMEM((N_TILES * 16,), i32),
            pltpu.VMEM_SHARED((N_TILES * 16,), i32),
            pltpu.SemaphoreType.DMA,
        ],
    )(_compact_kernel)
    cf = compact(
        key.reshape(N_PAD),
        vals.reshape(N_FIELDS * N_PAD),
        jnp.broadcast_to(thr[0, 0], (16,)),
        jnp.broadcast_to(thr[1, 0], (16,)),
    )

    boxes, misc = pl.pallas_call(
        _nms_kernel,
        in_specs=[pl.BlockSpec(memory_space=pltpu.MemorySpace.VMEM)]
        + [pl.BlockSpec(memory_space=pltpu.MemorySpace.SMEM)] * 5,
        out_shape=[
            jax.ShapeDtypeStruct((128, 4), f32),
            jax.ShapeDtypeStruct((128, 4), f32),
        ],
    )(jnp.stack([c.reshape(8, 128) for c in cf]),
      cf[0], cf[1], cf[2], cf[3], cf[5])

    det_boxes = boxes[:MAX_DET]
    det_ids = misc[:MAX_DET, 0].astype(i32)
    det_probs = misc[:MAX_DET, 1]
    det_valid = misc[:MAX_DET, 2] > 0.5
    det_boxes = jnp.where(det_valid[:, None], det_boxes, 0.0)
    return (det_boxes, det_ids, det_probs, det_valid)
